# baseline (speedup 1.0000x reference)
import functools

import jax
import jax.numpy as jnp
from jax import lax
from jax.experimental import pallas as pl
from jax.experimental.pallas import tpu as pltpu
from jax.experimental.pallas import tpu_sc as plsc

N_NODES = 10000
D_IN = 128
D_H = 128
N_CLS = 64

NC = 2
NS = 16
NW = NC * NS

CH = 128
K = 3


def _mlp_body(x_ref, w1_ref, b1_ref, w2_ref, b2_ref, z_ref):
    h = jnp.dot(x_ref[...], w1_ref[...], preferred_element_type=jnp.float32)
    h = jnp.maximum(h + b1_ref[...], 0.0)
    z_ref[...] = jnp.dot(h, w2_ref[...], preferred_element_type=jnp.float32) + b2_ref[...]


def _node_mlp(x, W1, b1, W2, b2):
    n = x.shape[0]
    blk = 2000
    grid = (n // blk,)
    return pl.pallas_call(
        _mlp_body,
        grid=grid,
        in_specs=[
            pl.BlockSpec((blk, D_IN), lambda i: (i, 0)),
            pl.BlockSpec((D_IN, D_H), lambda i: (0, 0)),
            pl.BlockSpec((1, D_H), lambda i: (0, 0)),
            pl.BlockSpec((D_H, N_CLS), lambda i: (0, 0)),
            pl.BlockSpec((1, N_CLS), lambda i: (0, 0)),
        ],
        out_specs=pl.BlockSpec((blk, N_CLS), lambda i: (i, 0)),
        out_shape=jax.ShapeDtypeStruct((n, N_CLS), jnp.float32),
    )(x, W1, b1.reshape(1, D_H), W2, b2.reshape(1, N_CLS))


def _gcn_gather(z, col):
    E = col.shape[0]
    d = z.shape[1]
    NBLK = E // CH
    BASE = NBLK // NW
    REM = NBLK - BASE * NW
    n_groups = BASE // K
    assert NBLK * CH == E and n_groups * K == BASE and REM < NW
    nt = d // 8

    mesh = plsc.VectorSubcoreMesh(core_axis_name="c", subcore_axis_name="s")

    @functools.partial(
        pl.kernel,
        mesh=mesh,
        compiler_params=pltpu.CompilerParams(
            use_tc_tiling_on_sc=False, needs_layout_passes=False),
        out_type=jax.ShapeDtypeStruct((E * d,), jnp.float32),
        scratch_types=[
            pltpu.VMEM_SHARED((N_NODES, d), jnp.float32),
            pltpu.VMEM((N_NODES,), jnp.int32),
            pltpu.VMEM(((BASE + 1) * CH,), jnp.int32),
            pltpu.VMEM((K, CH), jnp.int32),
            pltpu.VMEM((K, CH, d), jnp.float32),
            pltpu.VMEM((K, CH * d), jnp.float32),
            *([pltpu.SemaphoreType.DMA] * (2 * K)),
        ],
    )
    def k(z_hbm, col_hbm, out_hbm, zsh, colh_v, colw_v, ibuf, rbuf, tbuf, *sems):
        gsems, wsems = sems[:K], sems[K:]
        wid = lax.axis_index("s") * NC + lax.axis_index("c")
        start_blk = wid * BASE + jnp.minimum(wid, REM)
        base = start_blk * CH
        extra = wid < REM

        @pl.when(lax.axis_index("s") == 0)
        def _():
            pltpu.sync_copy(z_hbm, zsh)
        pltpu.sync_copy(col_hbm.at[pl.ds(0, N_NODES)], colh_v)
        pltpu.sync_copy(col_hbm.at[pl.ds(base, BASE * CH)],
                        colw_v.at[pl.ds(0, BASE * CH)])

        @pl.when(extra)
        def _():
            pltpu.sync_copy(col_hbm.at[pl.ds(base + BASE * CH, CH)],
                            colw_v.at[pl.ds(BASE * CH, CH)])
        plsc.subcore_barrier()

        ei = jnp.arange(16, dtype=jnp.int32)

        def compute_ibuf(slot, lb):
            for kk in range(CH // 16):
                cv = colw_v[pl.ds(lb * CH + kk * 16, 16)]
                ibuf[slot, pl.ds(kk * 16, 16)] = plsc.load_gather(colh_v, [cv])

        def transpose(slot):
            def f_step(f, carry):
                for e0 in range(CH // 16):
                    vals = plsc.load_gather(
                        rbuf.at[slot], [ei + e0 * 16, jnp.full((16,), f, jnp.int32)])
                    tbuf[slot, pl.ds(f * CH + e0 * 16, 16)] = vals
                return carry
            lax.fori_loop(0, d, f_step, 0)

        def fire_writes(slot, ec):
            for t in range(nt):
                pltpu.async_copy(
                    tbuf.at[slot, pl.ds(t * 8 * CH, 8 * CH)],
                    out_hbm.at[pl.ds((t * NBLK + ec) * 8 * CH, 8 * CH)],
                    wsems[slot])

        def drain_writes(slot):
            pltpu.make_async_copy(
                tbuf.at[slot], out_hbm.at[pl.ds(0, CH * d)], wsems[slot]).wait()

        def process(slot, lb, wait_write):
            @pl.when(wait_write)
            def _():
                drain_writes(slot)
            compute_ibuf(slot, lb)
            return pltpu.async_copy(zsh.at[ibuf.at[slot]], rbuf.at[slot], gsems[slot])

        def group(g, carry):
            handles = [process(b, g * K + b, g > 0) for b in range(K)]
            for b in range(K):
                handles[b].wait()
                transpose(b)
                fire_writes(b, start_blk + g * K + b)
            return carry

        lax.fori_loop(0, n_groups, group, 0)
        for b in range(K):
            drain_writes(b)

        @pl.when(extra)
        def _():
            h = process(0, BASE, False)
            h.wait()
            transpose(0)
            fire_writes(0, start_blk + BASE)
            drain_writes(0)

    return k(z, col)


def kernel(x, edge_index, W1, b1, W2, b2):
    col = edge_index[1]
    z = _node_mlp(x, W1, b1, W2, b2)
    flat = _gcn_gather(z, col)
    E = col.shape[0]
    a = flat.reshape(N_CLS // 8, E // CH, 8, CH)
    return a.transpose(1, 3, 0, 2).reshape(E, N_CLS)


# speedup vs baseline: 2.2489x; 2.2489x over previous
import functools

import jax
import jax.numpy as jnp
from jax import lax
from jax.experimental import pallas as pl
from jax.experimental.pallas import tpu as pltpu
from jax.experimental.pallas import tpu_sc as plsc

N_NODES = 10000
D_IN = 128
D_H = 128
N_CLS = 64

NC = 2
NS = 16
NW = NC * NS

CH = 128
K = 6


def _mlp_body(x_ref, w1_ref, b1_ref, w2_ref, b2_ref, zt_ref):
    h = jnp.dot(x_ref[...], w1_ref[...], preferred_element_type=jnp.float32)
    h = jnp.maximum(h + b1_ref[...], 0.0)
    z = jnp.dot(h, w2_ref[...], preferred_element_type=jnp.float32) + b2_ref[...]
    zt_ref[...] = z.T


def _node_mlp_t(x, W1, b1, W2, b2):
    n = x.shape[0]
    return pl.pallas_call(
        _mlp_body,
        out_shape=jax.ShapeDtypeStruct((N_CLS, n), jnp.float32),
    )(x, W1, b1.reshape(1, D_H), W2, b2.reshape(1, N_CLS))


def _gcn_gather(zt, col):
    E = col.shape[0]
    d = zt.shape[0]
    NBLK = E // CH
    BASE = NBLK // NW
    REM = NBLK - BASE * NW
    n_groups = BASE // K
    assert NBLK * CH == E and n_groups * K == BASE and REM < NW
    nt = d // 8

    mesh = plsc.VectorSubcoreMesh(core_axis_name="c", subcore_axis_name="s")

    @functools.partial(
        pl.kernel,
        mesh=mesh,
        compiler_params=pltpu.CompilerParams(
            use_tc_tiling_on_sc=False, needs_layout_passes=False),
        out_type=jax.ShapeDtypeStruct((E * d,), jnp.float32),
        scratch_types=[
            pltpu.VMEM_SHARED((d, N_NODES), jnp.float32),
            pltpu.VMEM((N_NODES,), jnp.int32),
            pltpu.VMEM(((BASE + 1) * CH,), jnp.int32),
            pltpu.VMEM((K, CH), jnp.int32),
            pltpu.VMEM((K, CH * d), jnp.float32),
            *([pltpu.SemaphoreType.DMA] * (2 * K)),
        ],
    )
    def k(zt_hbm, col_hbm, out_hbm, ztsh, colh_v, colw_v, ibuf, tbuf, *sems):
        gsems, wsems = sems[:K], sems[K:]
        wid = lax.axis_index("s") * NC + lax.axis_index("c")
        start_blk = wid * BASE + jnp.minimum(wid, REM)
        base = start_blk * CH
        extra = wid < REM

        @pl.when(lax.axis_index("s") == 0)
        def _():
            pltpu.sync_copy(zt_hbm, ztsh)
        pltpu.sync_copy(col_hbm.at[pl.ds(0, N_NODES)], colh_v)
        pltpu.sync_copy(col_hbm.at[pl.ds(base, BASE * CH)],
                        colw_v.at[pl.ds(0, BASE * CH)])

        @pl.when(extra)
        def _():
            pltpu.sync_copy(col_hbm.at[pl.ds(base + BASE * CH, CH)],
                            colw_v.at[pl.ds(BASE * CH, CH)])
        plsc.subcore_barrier()

        def compute_ibuf(slot, lb):
            for kk in range(CH // 16):
                cv = colw_v[pl.ds(lb * CH + kk * 16, 16)]
                ibuf[slot, pl.ds(kk * 16, 16)] = plsc.load_gather(colh_v, [cv])

        def fire_gathers(slot):
            def f_step(f, carry):
                pltpu.async_copy(
                    ztsh.at[f].at[ibuf.at[slot]],
                    tbuf.at[slot, pl.ds(f * CH, CH)],
                    gsems[slot])
                return carry
            lax.fori_loop(0, d, f_step, 0)

        def drain(slot, sem):
            pltpu.make_async_copy(
                tbuf.at[slot], out_hbm.at[pl.ds(0, CH * d)], sem).wait()

        def fire_writes(slot, ec):
            for t in range(nt):
                pltpu.async_copy(
                    tbuf.at[slot, pl.ds(t * 8 * CH, 8 * CH)],
                    out_hbm.at[pl.ds((t * NBLK + ec) * 8 * CH, 8 * CH)],
                    wsems[slot])

        def process(slot, lb, wait_write):
            @pl.when(wait_write)
            def _():
                drain(slot, wsems[slot])
            compute_ibuf(slot, lb)
            fire_gathers(slot)

        def group(g, carry):
            for b in range(K):
                process(b, g * K + b, g > 0)
            for b in range(K):
                drain(b, gsems[b])
                fire_writes(b, start_blk + g * K + b)
            return carry

        lax.fori_loop(0, n_groups, group, 0)
        for b in range(K):
            drain(b, wsems[b])

        @pl.when(extra)
        def _():
            process(0, BASE, False)
            drain(0, gsems[0])
            fire_writes(0, start_blk + BASE)
            drain(0, wsems[0])

    return k(zt, col)


def kernel(x, edge_index, W1, b1, W2, b2):
    col = edge_index[1]
    zt = _node_mlp_t(x, W1, b1, W2, b2)
    flat = _gcn_gather(zt, col)
    E = col.shape[0]
    a = flat.reshape(N_CLS // 8, E // CH, 8, CH)
    return a.transpose(1, 3, 0, 2).reshape(E, N_CLS)
